# baseline (speedup 1.0000x reference)
import jax
import jax.numpy as jnp
from jax import lax
from jax.experimental import pallas as pl
from jax.experimental.pallas import tpu as pltpu
from jax.experimental.pallas import tpu_sc as plsc

NC = 2
NS = 16
CH = 128
K = 16


def _mesh():
    return plsc.VectorSubcoreMesh(core_axis_name="c", subcore_axis_name="s")


def _make_prop_hbm(npad, n_super, F):
    P = NS
    rows_pt = npad // NS

    def body(tab_hbm, src_hbm, dst_hbm, zero_hbm, out_hbm,
             acc_sp, sidx, didx, msg, gsem, ssem):
        c = lax.axis_index("c")
        s = lax.axis_index("s")
        r0 = s * rows_pt
        pltpu.sync_copy(zero_hbm.at[pl.ds(r0, rows_pt)],
                        acc_sp.at[pl.ds(r0, rows_pt)])
        plsc.subcore_barrier()

        q = s
        cnt = (n_super - 1 - q) // P + 1

        def it(i, carry):
            row = (q + P * i) * K
            pltpu.sync_copy(src_hbm.at[pl.ds(row, K)], sidx)
            pltpu.sync_copy(dst_hbm.at[pl.ds(row, K)], didx)
            gds = [pltpu.async_copy(tab_hbm.at[c].at[sidx.at[j]],
                                    msg.at[j], gsem)
                   for j in range(K)]
            sds = []
            for j in range(K):
                gds[j].wait()
                sds.append(pltpu.async_copy(msg.at[j], acc_sp.at[didx.at[j]],
                                            ssem, add=True))
            for d in sds:
                d.wait()
            return carry

        lax.fori_loop(0, cnt, it, 0)
        plsc.subcore_barrier()
        pltpu.sync_copy(acc_sp.at[pl.ds(r0, rows_pt)],
                        out_hbm.at[c].at[pl.ds(r0, rows_pt)])

    return pl.kernel(
        body,
        out_type=jax.ShapeDtypeStruct((NC, npad, F), jnp.float32),
        mesh=_mesh(),
        compiler_params=pltpu.CompilerParams(use_tc_tiling_on_sc=False),
        scratch_types=[
            pltpu.VMEM_SHARED((npad, F), jnp.float32),
            pltpu.VMEM((K, CH), jnp.int32),
            pltpu.VMEM((K, CH), jnp.int32),
            pltpu.VMEM((K, CH, F), jnp.float32),
            pltpu.SemaphoreType.DMA,
            pltpu.SemaphoreType.DMA,
        ],
    )


def _make_prop_cols(npad, n_super, Fn):
    P = NC * NS
    rows_pt = npad // NS

    def body(tab_hbm, src_hbm, dst_hbm, zero_hbm, out_hbm,
             tab_sp, acc_sp, sidx, didx, msg, gsem, ssem):
        c = lax.axis_index("c")
        s = lax.axis_index("s")
        r0 = s * rows_pt
        for f in range(Fn):
            pltpu.sync_copy(tab_hbm.at[f].at[pl.ds(r0, rows_pt)],
                            tab_sp.at[f].at[pl.ds(r0, rows_pt)])
            pltpu.sync_copy(zero_hbm.at[f].at[pl.ds(r0, rows_pt)],
                            acc_sp.at[f].at[pl.ds(r0, rows_pt)])
        plsc.subcore_barrier()

        q = c * NS + s
        cnt = (n_super - 1 - q) // P + 1

        def it(i, carry):
            row = (q + P * i) * K
            pltpu.sync_copy(src_hbm.at[pl.ds(row, K)], sidx)
            pltpu.sync_copy(dst_hbm.at[pl.ds(row, K)], didx)
            gds = [pltpu.async_copy(tab_sp.at[f].at[sidx.at[j]],
                                    msg.at[f].at[j], gsem)
                   for j in range(K) for f in range(Fn)]
            sds = []
            d = 0
            for j in range(K):
                for f in range(Fn):
                    gds[d].wait()
                    d += 1
                    sds.append(pltpu.async_copy(
                        msg.at[f].at[j], acc_sp.at[f].at[didx.at[j]],
                        ssem, add=True))
            for dd in sds:
                dd.wait()
            return carry

        lax.fori_loop(0, cnt, it, 0)
        plsc.subcore_barrier()
        for f in range(Fn):
            pltpu.sync_copy(acc_sp.at[f].at[pl.ds(r0, rows_pt)],
                            out_hbm.at[c].at[f].at[pl.ds(r0, rows_pt)])

    return pl.kernel(
        body,
        out_type=jax.ShapeDtypeStruct((NC, Fn, npad), jnp.float32),
        mesh=_mesh(),
        compiler_params=pltpu.CompilerParams(use_tc_tiling_on_sc=False),
        scratch_types=[
            pltpu.VMEM_SHARED((Fn, npad), jnp.float32),
            pltpu.VMEM_SHARED((Fn, npad), jnp.float32),
            pltpu.VMEM((K, CH), jnp.int32),
            pltpu.VMEM((K, CH), jnp.int32),
            pltpu.VMEM((Fn, K, CH), jnp.float32),
            pltpu.SemaphoreType.DMA,
            pltpu.SemaphoreType.DMA,
        ],
    )


def _make_hist(npad, n_super):
    P = NC * NS
    rows_pt = npad // NS

    def body(dst_hbm, zero_hbm, ones_hbm, out_hbm, acc_sp, didx, ones_v, ssem):
        c = lax.axis_index("c")
        s = lax.axis_index("s")
        r0 = s * rows_pt
        pltpu.sync_copy(zero_hbm.at[pl.ds(r0, rows_pt)],
                        acc_sp.at[pl.ds(r0, rows_pt)])
        pltpu.sync_copy(ones_hbm, ones_v)
        plsc.subcore_barrier()

        q = c * NS + s
        cnt = (n_super - 1 - q) // P + 1

        def it(i, carry):
            row = (q + P * i) * K
            pltpu.sync_copy(dst_hbm.at[pl.ds(row, K)], didx)
            sds = [pltpu.async_copy(ones_v, acc_sp.at[didx.at[j]], ssem,
                                    add=True)
                   for j in range(K)]
            for d in sds:
                d.wait()
            return carry

        lax.fori_loop(0, cnt, it, 0)
        plsc.subcore_barrier()
        pltpu.sync_copy(acc_sp.at[pl.ds(r0, rows_pt)],
                        out_hbm.at[c].at[pl.ds(r0, rows_pt)])

    return pl.kernel(
        body,
        out_type=jax.ShapeDtypeStruct((NC, npad), jnp.float32),
        mesh=_mesh(),
        compiler_params=pltpu.CompilerParams(use_tc_tiling_on_sc=False),
        scratch_types=[
            pltpu.VMEM_SHARED((npad,), jnp.float32),
            pltpu.VMEM((K, CH), jnp.int32),
            pltpu.VMEM((CH,), jnp.float32),
            pltpu.SemaphoreType.DMA,
        ],
    )


def _tc1(hist, xT):
    npad = xT.shape[1]

    def body(hist_ref, xT_ref, dinv_ref, g0_ref):
        deg = hist_ref[0:1] + hist_ref[1:2] + 1.0
        dinv = lax.rsqrt(deg)
        dinv_ref[...] = dinv
        g0_ref[...] = xT_ref[...] * dinv

    return pl.pallas_call(
        body,
        out_shape=[jax.ShapeDtypeStruct((1, npad), jnp.float32),
                   jax.ShapeDtypeStruct((2, npad), jnp.float32)],
    )(hist, xT)


def _tc2(s0, g0, dinv, W1, b1, W2):
    npad = g0.shape[1]

    def body(s0_ref, g0_ref, dinv_ref, W1_ref, b1_ref, W2_ref, g1_ref):
        p1 = (s0_ref[0] + s0_ref[1] + g0_ref[...]) * dinv_ref[...]
        W1v = W1_ref[...]
        h1 = (W1v[0][:, None] * p1[0:1] + W1v[1][:, None] * p1[1:2]
              + b1_ref[...][:, None])
        h1 = jnp.maximum(h1, 0.0)
        g1 = lax.dot_general(W2_ref[...], h1, (((0,), (0,)), ((), ())),
                             preferred_element_type=jnp.float32)
        g1_ref[...] = g1 * dinv_ref[...]

    return pl.pallas_call(
        body,
        out_shape=jax.ShapeDtypeStruct((16, npad), jnp.float32),
    )(s0, g0, dinv, W1, b1, W2)


def _tc3(s1T, g1, dinv, b2, W3):
    npad = g1.shape[1]

    def body(s1_ref, g1_ref, dinv_ref, b2_ref, W3_ref, g2_ref):
        p2 = (s1_ref[...] + g1_ref[...]) * dinv_ref[...]
        h2 = jnp.maximum(p2 + b2_ref[...][:, None], 0.0)
        g2 = jnp.sum(h2 * W3_ref[...], axis=0, keepdims=True)
        g2_ref[...] = g2 * dinv_ref[...]

    return pl.pallas_call(
        body,
        out_shape=jax.ShapeDtypeStruct((1, npad), jnp.float32),
    )(s1T, g1, dinv, b2, W3)


def _tc4(s2, g2, dinv, b3):
    npad = g2.shape[1]

    def body(s2_ref, g2_ref, dinv_ref, b3_ref, out_ref):
        out_ref[...] = ((s2_ref[0] + s2_ref[1] + g2_ref[...])
                        * dinv_ref[...] + b3_ref[...])

    return pl.pallas_call(
        body,
        out_shape=jax.ShapeDtypeStruct((1, npad), jnp.float32),
    )(s2, g2, dinv, b3)


def kernel(x, edge_index, W1, b1, W2, b2, W3, b3):
    n = x.shape[0]
    e = edge_index.shape[1]
    npad = -(-n // (NS * CH)) * (NS * CH)
    n_super = e // (CH * K)

    src2d = edge_index[0].reshape(e // CH, CH)
    dst2d = edge_index[1].reshape(e // CH, CH)

    pad_n = npad - n
    zc1 = jnp.zeros((1, npad), jnp.float32)
    zc2 = jnp.zeros((2, npad), jnp.float32)
    zr8 = jnp.zeros((npad, 8), jnp.float32)
    zh = jnp.zeros((npad,), jnp.float32)
    ones = jnp.ones((CH,), jnp.float32)

    xT = jnp.pad(x, ((0, pad_n), (0, 0))).T

    hist = _make_hist(npad, n_super)(dst2d, zh, ones)
    dinv, g0 = _tc1(hist, xT)

    s0 = _make_prop_cols(npad, n_super, 2)(g0, src2d, dst2d, zc2)
    g1 = _tc2(s0, g0, dinv, W1, b1, W2)

    g1rows = g1.reshape(2, 8, npad).transpose(0, 2, 1)
    s1 = _make_prop_hbm(npad, n_super, 8)(g1rows, src2d, dst2d, zr8)
    g2 = _tc3(s1.transpose(0, 2, 1).reshape(16, npad), g1, dinv, b2, W3)

    s2 = _make_prop_cols(npad, n_super, 1)(g2, src2d, dst2d, zc1)
    outT = _tc4(s2, g2, dinv, b3)
    return outT[0, :n, None]


# speedup vs baseline: 1.1000x; 1.1000x over previous
import jax
import jax.numpy as jnp
from jax import lax
from jax.experimental import pallas as pl
from jax.experimental.pallas import tpu as pltpu
from jax.experimental.pallas import tpu_sc as plsc

NC = 2
NS = 16
CH = 128
K = 16


def _mesh():
    return plsc.VectorSubcoreMesh(core_axis_name="c", subcore_axis_name="s")


def _sc_params():
    return pltpu.CompilerParams(use_tc_tiling_on_sc=False)


def _make_prop8(npad, n_super, feat_split):
    P = NS if feat_split else NC * NS
    rows_pt = npad // NS
    F = 8

    def body(tab_hbm, src_hbm, dst_hbm, zero_hbm, out_hbm,
             tab_sp, acc_sp, sidx, didx, msg, gsem, ssem):
        c = lax.axis_index("c")
        s = lax.axis_index("s")
        r0 = s * rows_pt
        tsrc = tab_hbm.at[c] if feat_split else tab_hbm
        pltpu.sync_copy(tsrc.at[pl.ds(r0, rows_pt)],
                        tab_sp.at[pl.ds(r0, rows_pt)])
        pltpu.sync_copy(zero_hbm.at[pl.ds(r0, rows_pt)],
                        acc_sp.at[pl.ds(r0, rows_pt)])
        plsc.subcore_barrier()

        q = s if feat_split else c * NS + s
        cnt = (n_super - 1 - q) // P + 1

        pltpu.sync_copy(src_hbm.at[pl.ds(q * K, K)], sidx.at[0])
        pltpu.sync_copy(dst_hbm.at[pl.ds(q * K, K)], didx.at[0])

        def it2(i2, carry):
            for b in range(2):
                i = 2 * i2 + b

                @pl.when(i < cnt)
                def _():
                    gds = [pltpu.async_copy(tab_sp.at[sidx.at[b].at[j]],
                                            msg.at[j], gsem)
                           for j in range(K)]

                    @pl.when(i + 1 < cnt)
                    def _():
                        rown = (q + P * (i + 1)) * K
                        pltpu.sync_copy(src_hbm.at[pl.ds(rown, K)],
                                        sidx.at[1 - b])
                        pltpu.sync_copy(dst_hbm.at[pl.ds(rown, K)],
                                        didx.at[1 - b])

                    sds = []
                    for j in range(K):
                        gds[j].wait()
                        sds.append(pltpu.async_copy(
                            msg.at[j], acc_sp.at[didx.at[b].at[j]],
                            ssem, add=True))
                    for d in sds:
                        d.wait()
            return carry

        lax.fori_loop(0, (cnt + 1) // 2, it2, 0)
        plsc.subcore_barrier()
        pltpu.sync_copy(acc_sp.at[pl.ds(r0, rows_pt)],
                        out_hbm.at[c].at[pl.ds(r0, rows_pt)])

    return pl.kernel(
        body,
        out_type=jax.ShapeDtypeStruct((NC, npad, F), jnp.float32),
        mesh=_mesh(),
        compiler_params=_sc_params(),
        scratch_types=[
            pltpu.VMEM_SHARED((npad, F), jnp.float32),
            pltpu.VMEM_SHARED((npad, F), jnp.float32),
            pltpu.VMEM((2, K, CH), jnp.int32),
            pltpu.VMEM((2, K, CH), jnp.int32),
            pltpu.VMEM((K, CH, F), jnp.float32),
            pltpu.SemaphoreType.DMA,
            pltpu.SemaphoreType.DMA,
        ],
    )


def _make_prop_col(npad, n_super):
    P = NC * NS
    rows_pt = npad // NS

    def body(tab_hbm, src_hbm, dst_hbm, zero_hbm, out_hbm,
             tab_sp, acc_sp, sidx, didx, msg, gsem, ssem):
        c = lax.axis_index("c")
        s = lax.axis_index("s")
        r0 = s * rows_pt
        pltpu.sync_copy(tab_hbm.at[0].at[pl.ds(r0, rows_pt)],
                        tab_sp.at[pl.ds(r0, rows_pt)])
        pltpu.sync_copy(zero_hbm.at[0].at[pl.ds(r0, rows_pt)],
                        acc_sp.at[pl.ds(r0, rows_pt)])
        plsc.subcore_barrier()

        q = c * NS + s
        cnt = (n_super - 1 - q) // P + 1

        pltpu.sync_copy(src_hbm.at[pl.ds(q * K, K)], sidx.at[0])
        pltpu.sync_copy(dst_hbm.at[pl.ds(q * K, K)], didx.at[0])

        def it2(i2, carry):
            for b in range(2):
                i = 2 * i2 + b

                @pl.when(i < cnt)
                def _():
                    gds = [pltpu.async_copy(tab_sp.at[sidx.at[b].at[j]],
                                            msg.at[j], gsem)
                           for j in range(K)]

                    @pl.when(i + 1 < cnt)
                    def _():
                        rown = (q + P * (i + 1)) * K
                        pltpu.sync_copy(src_hbm.at[pl.ds(rown, K)],
                                        sidx.at[1 - b])
                        pltpu.sync_copy(dst_hbm.at[pl.ds(rown, K)],
                                        didx.at[1 - b])

                    sds = []
                    for j in range(K):
                        gds[j].wait()
                        sds.append(pltpu.async_copy(
                            msg.at[j], acc_sp.at[didx.at[b].at[j]],
                            ssem, add=True))
                    for d in sds:
                        d.wait()
            return carry

        lax.fori_loop(0, (cnt + 1) // 2, it2, 0)
        plsc.subcore_barrier()
        pltpu.sync_copy(acc_sp.at[pl.ds(r0, rows_pt)],
                        out_hbm.at[c].at[0].at[pl.ds(r0, rows_pt)])

    return pl.kernel(
        body,
        out_type=jax.ShapeDtypeStruct((NC, 1, npad), jnp.float32),
        mesh=_mesh(),
        compiler_params=_sc_params(),
        scratch_types=[
            pltpu.VMEM_SHARED((npad,), jnp.float32),
            pltpu.VMEM_SHARED((npad,), jnp.float32),
            pltpu.VMEM((2, K, CH), jnp.int32),
            pltpu.VMEM((2, K, CH), jnp.int32),
            pltpu.VMEM((K, CH), jnp.float32),
            pltpu.SemaphoreType.DMA,
            pltpu.SemaphoreType.DMA,
        ],
    )


def _make_hist(npad, n_super):
    P = NC * NS
    rows_pt = npad // NS

    def body(dst_hbm, zero_hbm, ones_hbm, out_hbm, acc_sp, didx, ones_v, ssem):
        c = lax.axis_index("c")
        s = lax.axis_index("s")
        r0 = s * rows_pt
        pltpu.sync_copy(zero_hbm.at[pl.ds(r0, rows_pt)],
                        acc_sp.at[pl.ds(r0, rows_pt)])
        pltpu.sync_copy(ones_hbm, ones_v)
        plsc.subcore_barrier()

        q = c * NS + s
        cnt = (n_super - 1 - q) // P + 1

        pltpu.sync_copy(dst_hbm.at[pl.ds(q * K, K)], didx.at[0])

        def it2(i2, carry):
            for b in range(2):
                i = 2 * i2 + b

                @pl.when(i < cnt)
                def _():
                    sds = [pltpu.async_copy(ones_v,
                                            acc_sp.at[didx.at[b].at[j]],
                                            ssem, add=True)
                           for j in range(K)]

                    @pl.when(i + 1 < cnt)
                    def _():
                        rown = (q + P * (i + 1)) * K
                        pltpu.sync_copy(dst_hbm.at[pl.ds(rown, K)],
                                        didx.at[1 - b])

                    for d in sds:
                        d.wait()
            return carry

        lax.fori_loop(0, (cnt + 1) // 2, it2, 0)
        plsc.subcore_barrier()
        pltpu.sync_copy(acc_sp.at[pl.ds(r0, rows_pt)],
                        out_hbm.at[c].at[pl.ds(r0, rows_pt)])

    return pl.kernel(
        body,
        out_type=jax.ShapeDtypeStruct((NC, npad), jnp.float32),
        mesh=_mesh(),
        compiler_params=_sc_params(),
        scratch_types=[
            pltpu.VMEM_SHARED((npad,), jnp.float32),
            pltpu.VMEM((2, K, CH), jnp.int32),
            pltpu.VMEM((CH,), jnp.float32),
            pltpu.SemaphoreType.DMA,
        ],
    )


def _tc1(hist, xT):
    npad = xT.shape[1]

    def body(hist_ref, xT_ref, dinv_ref, g0_ref):
        deg = hist_ref[0:1] + hist_ref[1:2] + 1.0
        dinv = lax.rsqrt(deg)
        dinv_ref[...] = dinv
        g0_ref[...] = xT_ref[...] * dinv

    return pl.pallas_call(
        body,
        out_shape=[jax.ShapeDtypeStruct((1, npad), jnp.float32),
                   jax.ShapeDtypeStruct((2, npad), jnp.float32)],
    )(hist, xT)


def _tc2(s0T, g0, dinv, W1, b1, W2):
    npad = g0.shape[1]

    def body(s0_ref, g0_ref, dinv_ref, W1_ref, b1_ref, W2_ref, g1_ref):
        p1 = (s0_ref[...] + g0_ref[...]) * dinv_ref[...]
        W1v = W1_ref[...]
        h1 = (W1v[0][:, None] * p1[0:1] + W1v[1][:, None] * p1[1:2]
              + b1_ref[...][:, None])
        h1 = jnp.maximum(h1, 0.0)
        g1 = lax.dot_general(W2_ref[...], h1, (((0,), (0,)), ((), ())),
                             preferred_element_type=jnp.float32)
        g1_ref[...] = g1 * dinv_ref[...]

    return pl.pallas_call(
        body,
        out_shape=jax.ShapeDtypeStruct((16, npad), jnp.float32),
    )(s0T, g0, dinv, W1, b1, W2)


def _tc3(s1T, g1, dinv, b2, W3):
    npad = g1.shape[1]

    def body(s1_ref, g1_ref, dinv_ref, b2_ref, W3_ref, g2_ref):
        p2 = (s1_ref[...] + g1_ref[...]) * dinv_ref[...]
        h2 = jnp.maximum(p2 + b2_ref[...][:, None], 0.0)
        g2 = jnp.sum(h2 * W3_ref[...], axis=0, keepdims=True)
        g2_ref[...] = g2 * dinv_ref[...]

    return pl.pallas_call(
        body,
        out_shape=jax.ShapeDtypeStruct((1, npad), jnp.float32),
    )(s1T, g1, dinv, b2, W3)


def _tc4(s2, g2, dinv, b3):
    npad = g2.shape[1]

    def body(s2_ref, g2_ref, dinv_ref, b3_ref, out_ref):
        out_ref[...] = ((s2_ref[0] + s2_ref[1] + g2_ref[...])
                        * dinv_ref[...] + b3_ref[...])

    return pl.pallas_call(
        body,
        out_shape=jax.ShapeDtypeStruct((1, npad), jnp.float32),
    )(s2, g2, dinv, b3)


def kernel(x, edge_index, W1, b1, W2, b2, W3, b3):
    n = x.shape[0]
    e = edge_index.shape[1]
    npad = -(-n // (NS * CH)) * (NS * CH)
    n_super = e // (CH * K)

    src2d = edge_index[0].reshape(e // CH, CH)
    dst2d = edge_index[1].reshape(e // CH, CH)

    pad_n = npad - n
    zc1 = jnp.zeros((1, npad), jnp.float32)
    zr8 = jnp.zeros((npad, 8), jnp.float32)
    zh = jnp.zeros((npad,), jnp.float32)
    ones = jnp.ones((CH,), jnp.float32)

    xT = jnp.pad(x, ((0, pad_n), (0, 0))).T

    hist = _make_hist(npad, n_super)(dst2d, zh, ones)
    dinv, g0 = _tc1(hist, xT)

    g0rows = jnp.concatenate(
        [g0.T, jnp.zeros((npad, 6), jnp.float32)], axis=1)
    s0 = _make_prop8(npad, n_super, False)(g0rows, src2d, dst2d, zr8)
    s0T = (s0[0] + s0[1]).T[:2]
    g1 = _tc2(s0T, g0, dinv, W1, b1, W2)

    g1rows = g1.reshape(2, 8, npad).transpose(0, 2, 1)
    s1 = _make_prop8(npad, n_super, True)(g1rows, src2d, dst2d, zr8)
    g2 = _tc3(s1.transpose(0, 2, 1).reshape(16, npad), g1, dinv, b2, W3)

    s2 = _make_prop_col(npad, n_super)(g2, src2d, dst2d, zc1)
    outT = _tc4(s2, g2, dinv, b3)
    return outT[0, :n, None]


# speedup vs baseline: 1.3399x; 1.2181x over previous
import jax
import jax.numpy as jnp
from jax import lax
from jax.experimental import pallas as pl
from jax.experimental.pallas import tpu as pltpu
from jax.experimental.pallas import tpu_sc as plsc

NC = 2
NS = 16
CH = 128
K = 16


def _mesh():
    return plsc.VectorSubcoreMesh(core_axis_name="c", subcore_axis_name="s")


def _sc_params():
    return pltpu.CompilerParams(use_tc_tiling_on_sc=False)


def _make_prop8(npad, n_super):
    P = NS
    rows_pt = npad // NS
    F = 8

    def body(tab_hbm, src_hbm, dst_hbm, zero_hbm, out_hbm,
             tab_sp, acc_sp, sidx, didx, msg, gsem, ssem, isem):
        c = lax.axis_index("c")
        s = lax.axis_index("s")
        r0 = s * rows_pt
        pltpu.sync_copy(tab_hbm.at[c].at[pl.ds(r0, rows_pt)],
                        tab_sp.at[pl.ds(r0, rows_pt)])
        pltpu.sync_copy(zero_hbm.at[pl.ds(r0, rows_pt)],
                        acc_sp.at[pl.ds(r0, rows_pt)])
        plsc.subcore_barrier()

        q = s
        cnt = (n_super - 1 - q) // P + 1

        pltpu.async_copy(src_hbm.at[pl.ds(q * K, K)], sidx.at[0], isem)
        pltpu.async_copy(dst_hbm.at[pl.ds(q * K, K)], didx.at[0], isem)

        def it2(i2, carry):
            for b in range(2):
                i = 2 * i2 + b

                @pl.when(i < cnt)
                def _():
                    pltpu.make_async_copy(src_hbm.at[pl.ds(0, K)],
                                          sidx.at[b], isem).wait()
                    pltpu.make_async_copy(dst_hbm.at[pl.ds(0, K)],
                                          didx.at[b], isem).wait()
                    gds = [pltpu.async_copy(tab_sp.at[sidx.at[b].at[j]],
                                            msg.at[j], gsem)
                           for j in range(K)]

                    @pl.when(i + 1 < cnt)
                    def _():
                        rown = (q + P * (i + 1)) * K
                        pltpu.async_copy(src_hbm.at[pl.ds(rown, K)],
                                         sidx.at[1 - b], isem)
                        pltpu.async_copy(dst_hbm.at[pl.ds(rown, K)],
                                         didx.at[1 - b], isem)

                    sds = []
                    for j in range(K):
                        gds[j].wait()
                        sds.append(pltpu.async_copy(
                            msg.at[j], acc_sp.at[didx.at[b].at[j]],
                            ssem, add=True))
                    for d in sds:
                        d.wait()
            return carry

        lax.fori_loop(0, (cnt + 1) // 2, it2, 0)
        plsc.subcore_barrier()
        pltpu.sync_copy(acc_sp.at[pl.ds(r0, rows_pt)],
                        out_hbm.at[c].at[pl.ds(r0, rows_pt)])

    return pl.kernel(
        body,
        out_type=jax.ShapeDtypeStruct((NC, npad, F), jnp.float32),
        mesh=_mesh(),
        compiler_params=_sc_params(),
        scratch_types=[
            pltpu.VMEM_SHARED((npad, F), jnp.float32),
            pltpu.VMEM_SHARED((npad, F), jnp.float32),
            pltpu.VMEM((2, K, CH), jnp.int32),
            pltpu.VMEM((2, K, CH), jnp.int32),
            pltpu.VMEM((K, CH, F), jnp.float32),
            pltpu.SemaphoreType.DMA,
            pltpu.SemaphoreType.DMA,
            pltpu.SemaphoreType.DMA,
        ],
    )


def _make_prop_cols(npad, n_super, Fn):
    P = NC * NS
    rows_pt = npad // NS

    def body(tab_hbm, src_hbm, dst_hbm, zero_hbm, out_hbm,
             tab_sp, acc_sp, sidx, didx, msg, gsem, ssem, isem):
        c = lax.axis_index("c")
        s = lax.axis_index("s")
        r0 = s * rows_pt
        for f in range(Fn):
            pltpu.sync_copy(tab_hbm.at[f].at[pl.ds(r0, rows_pt)],
                            tab_sp.at[f].at[pl.ds(r0, rows_pt)])
            pltpu.sync_copy(zero_hbm.at[f].at[pl.ds(r0, rows_pt)],
                            acc_sp.at[f].at[pl.ds(r0, rows_pt)])
        plsc.subcore_barrier()

        q = c * NS + s
        cnt = (n_super - 1 - q) // P + 1

        pltpu.async_copy(src_hbm.at[pl.ds(q * K, K)], sidx.at[0], isem)
        pltpu.async_copy(dst_hbm.at[pl.ds(q * K, K)], didx.at[0], isem)

        def it2(i2, carry):
            for b in range(2):
                i = 2 * i2 + b

                @pl.when(i < cnt)
                def _():
                    pltpu.make_async_copy(src_hbm.at[pl.ds(0, K)],
                                          sidx.at[b], isem).wait()
                    pltpu.make_async_copy(dst_hbm.at[pl.ds(0, K)],
                                          didx.at[b], isem).wait()
                    gds = [pltpu.async_copy(tab_sp.at[f].at[sidx.at[b].at[j]],
                                            msg.at[f].at[j], gsem)
                           for j in range(K) for f in range(Fn)]

                    @pl.when(i + 1 < cnt)
                    def _():
                        rown = (q + P * (i + 1)) * K
                        pltpu.async_copy(src_hbm.at[pl.ds(rown, K)],
                                         sidx.at[1 - b], isem)
                        pltpu.async_copy(dst_hbm.at[pl.ds(rown, K)],
                                         didx.at[1 - b], isem)

                    sds = []
                    d = 0
                    for j in range(K):
                        for f in range(Fn):
                            gds[d].wait()
                            d += 1
                            sds.append(pltpu.async_copy(
                                msg.at[f].at[j],
                                acc_sp.at[f].at[didx.at[b].at[j]],
                                ssem, add=True))
                    for dd in sds:
                        dd.wait()
            return carry

        lax.fori_loop(0, (cnt + 1) // 2, it2, 0)
        plsc.subcore_barrier()
        for f in range(Fn):
            pltpu.sync_copy(acc_sp.at[f].at[pl.ds(r0, rows_pt)],
                            out_hbm.at[c].at[f].at[pl.ds(r0, rows_pt)])

    return pl.kernel(
        body,
        out_type=jax.ShapeDtypeStruct((NC, Fn, npad), jnp.float32),
        mesh=_mesh(),
        compiler_params=_sc_params(),
        scratch_types=[
            pltpu.VMEM_SHARED((Fn, npad), jnp.float32),
            pltpu.VMEM_SHARED((Fn, npad), jnp.float32),
            pltpu.VMEM((2, K, CH), jnp.int32),
            pltpu.VMEM((2, K, CH), jnp.int32),
            pltpu.VMEM((Fn, K, CH), jnp.float32),
            pltpu.SemaphoreType.DMA,
            pltpu.SemaphoreType.DMA,
            pltpu.SemaphoreType.DMA,
        ],
    )


def _make_hist(npad, n_super):
    P = NC * NS
    rows_pt = npad // NS

    def body(dst_hbm, zero_hbm, ones_hbm, out_hbm, acc_sp, didx, ones_v,
             ssem, isem):
        c = lax.axis_index("c")
        s = lax.axis_index("s")
        r0 = s * rows_pt
        pltpu.sync_copy(zero_hbm.at[pl.ds(r0, rows_pt)],
                        acc_sp.at[pl.ds(r0, rows_pt)])
        pltpu.sync_copy(ones_hbm, ones_v)
        plsc.subcore_barrier()

        q = c * NS + s
        cnt = (n_super - 1 - q) // P + 1

        pltpu.async_copy(dst_hbm.at[pl.ds(q * K, K)], didx.at[0], isem)

        def it2(i2, carry):
            for b in range(2):
                i = 2 * i2 + b

                @pl.when(i < cnt)
                def _():
                    pltpu.make_async_copy(dst_hbm.at[pl.ds(0, K)],
                                          didx.at[b], isem).wait()
                    sds = [pltpu.async_copy(ones_v,
                                            acc_sp.at[didx.at[b].at[j]],
                                            ssem, add=True)
                           for j in range(K)]

                    @pl.when(i + 1 < cnt)
                    def _():
                        rown = (q + P * (i + 1)) * K
                        pltpu.async_copy(dst_hbm.at[pl.ds(rown, K)],
                                         didx.at[1 - b], isem)

                    for d in sds:
                        d.wait()
            return carry

        lax.fori_loop(0, (cnt + 1) // 2, it2, 0)
        plsc.subcore_barrier()
        pltpu.sync_copy(acc_sp.at[pl.ds(r0, rows_pt)],
                        out_hbm.at[c].at[pl.ds(r0, rows_pt)])

    return pl.kernel(
        body,
        out_type=jax.ShapeDtypeStruct((NC, npad), jnp.float32),
        mesh=_mesh(),
        compiler_params=_sc_params(),
        scratch_types=[
            pltpu.VMEM_SHARED((npad,), jnp.float32),
            pltpu.VMEM((2, K, CH), jnp.int32),
            pltpu.VMEM((CH,), jnp.float32),
            pltpu.SemaphoreType.DMA,
            pltpu.SemaphoreType.DMA,
        ],
    )


def _tc1(hist, xT):
    npad = xT.shape[1]

    def body(hist_ref, xT_ref, dinv_ref, g0_ref):
        deg = hist_ref[0:1] + hist_ref[1:2] + 1.0
        dinv = lax.rsqrt(deg)
        dinv_ref[...] = dinv
        g0_ref[...] = xT_ref[...] * dinv

    return pl.pallas_call(
        body,
        out_shape=[jax.ShapeDtypeStruct((1, npad), jnp.float32),
                   jax.ShapeDtypeStruct((2, npad), jnp.float32)],
    )(hist, xT)


def _tc2(s0, g0, dinv, W1, b1, W2):
    npad = g0.shape[1]

    def body(s0_ref, g0_ref, dinv_ref, W1_ref, b1_ref, W2_ref, g1_ref):
        p1 = (s0_ref[0] + s0_ref[1] + g0_ref[...]) * dinv_ref[...]
        W1v = W1_ref[...]
        h1 = (W1v[0][:, None] * p1[0:1] + W1v[1][:, None] * p1[1:2]
              + b1_ref[...][:, None])
        h1 = jnp.maximum(h1, 0.0)
        g1 = lax.dot_general(W2_ref[...], h1, (((0,), (0,)), ((), ())),
                             preferred_element_type=jnp.float32)
        g1_ref[...] = g1 * dinv_ref[...]

    return pl.pallas_call(
        body,
        out_shape=jax.ShapeDtypeStruct((16, npad), jnp.float32),
    )(s0, g0, dinv, W1, b1, W2)


def _tc3(s1T, g1, dinv, b2, W3):
    npad = g1.shape[1]

    def body(s1_ref, g1_ref, dinv_ref, b2_ref, W3_ref, g2_ref):
        p2 = (s1_ref[...] + g1_ref[...]) * dinv_ref[...]
        h2 = jnp.maximum(p2 + b2_ref[...][:, None], 0.0)
        g2 = jnp.sum(h2 * W3_ref[...], axis=0, keepdims=True)
        g2_ref[...] = g2 * dinv_ref[...]

    return pl.pallas_call(
        body,
        out_shape=jax.ShapeDtypeStruct((1, npad), jnp.float32),
    )(s1T, g1, dinv, b2, W3)


def _tc4(s2, g2, dinv, b3):
    npad = g2.shape[1]

    def body(s2_ref, g2_ref, dinv_ref, b3_ref, out_ref):
        out_ref[...] = ((s2_ref[0] + s2_ref[1] + g2_ref[...])
                        * dinv_ref[...] + b3_ref[...])

    return pl.pallas_call(
        body,
        out_shape=jax.ShapeDtypeStruct((1, npad), jnp.float32),
    )(s2, g2, dinv, b3)


def kernel(x, edge_index, W1, b1, W2, b2, W3, b3):
    n = x.shape[0]
    e = edge_index.shape[1]
    npad = -(-n // (NS * CH)) * (NS * CH)
    n_super = e // (CH * K)

    src2d = edge_index[0].reshape(e // CH, CH)
    dst2d = edge_index[1].reshape(e // CH, CH)

    pad_n = npad - n
    zc1 = jnp.zeros((1, npad), jnp.float32)
    zc2 = jnp.zeros((2, npad), jnp.float32)
    zr8 = jnp.zeros((npad, 8), jnp.float32)
    zh = jnp.zeros((npad,), jnp.float32)
    ones = jnp.ones((CH,), jnp.float32)

    xT = jnp.pad(x, ((0, pad_n), (0, 0))).T

    hist = _make_hist(npad, n_super)(dst2d, zh, ones)
    dinv, g0 = _tc1(hist, xT)

    s0 = _make_prop_cols(npad, n_super, 2)(g0, src2d, dst2d, zc2)
    g1 = _tc2(s0, g0, dinv, W1, b1, W2)

    g1rows = g1.reshape(2, 8, npad).transpose(0, 2, 1)
    s1 = _make_prop8(npad, n_super)(g1rows, src2d, dst2d, zr8)
    g2 = _tc3(s1.transpose(0, 2, 1).reshape(16, npad), g1, dinv, b2, W3)

    s2 = _make_prop_cols(npad, n_super, 1)(g2, src2d, dst2d, zc1)
    outT = _tc4(s2, g2, dinv, b3)
    return outT[0, :n, None]
